# baseline (speedup 1.0000x reference)
import jax
import jax.numpy as jnp
from jax.experimental import pallas as pl
from jax.experimental.pallas import tpu as pltpu

_R, _C = 128, 100000
_BC = 2048
_NB = pl.cdiv(_C, _BC)

_G_CACHE = None


def _gumbel_const():
    global _G_CACHE
    if _G_CACHE is None:
        _G_CACHE = jax.block_until_ready(
            jax.random.gumbel(jax.random.key(42), (_R, _C), jnp.float32))
    return _G_CACHE


def _argmax_body(x_ref, g_ref, idx_ref, m_ref):
    j = pl.program_id(0)

    @pl.when(j == 0)
    def _():
        m_ref[:] = jnp.full((_R, 1), -jnp.inf, jnp.float32)
        idx_ref[:] = jnp.zeros((_R, 1), jnp.int32)

    x = x_ref[:] + g_ref[:]
    cols = jax.lax.broadcasted_iota(jnp.int32, (_R, _BC), 1) + j * _BC
    x = jnp.where(cols < _C, x, -jnp.inf)
    bm = jnp.max(x, axis=1, keepdims=True)
    ba = jnp.min(jnp.where(x == bm, cols, _C), axis=1, keepdims=True)
    better = bm > m_ref[:]
    idx_ref[:] = jnp.where(better, ba, idx_ref[:]).astype(jnp.int32)
    m_ref[:] = jnp.where(better, bm, m_ref[:])


def _onehot_body(idx_ref, o_ref):
    j = pl.program_id(0)
    cols = jax.lax.broadcasted_iota(jnp.int32, (_R, _BC), 1) + j * _BC
    o_ref[:] = (cols == idx_ref[:]).astype(jnp.float32)


def kernel(logits):
    g = _gumbel_const()
    idx = pl.pallas_call(
        _argmax_body,
        grid=(_NB,),
        in_specs=[pl.BlockSpec((_R, _BC), lambda j: (0, j)),
                  pl.BlockSpec((_R, _BC), lambda j: (0, j))],
        out_specs=pl.BlockSpec((_R, 1), lambda j: (0, 0)),
        out_shape=jax.ShapeDtypeStruct((_R, 1), jnp.int32),
        scratch_shapes=[pltpu.VMEM((_R, 1), jnp.float32)],
    )(logits, g)
    out = pl.pallas_call(
        _onehot_body,
        grid=(_NB,),
        in_specs=[pl.BlockSpec((_R, 1), lambda j: (0, 0))],
        out_specs=pl.BlockSpec((_R, _BC), lambda j: (0, j)),
        out_shape=jax.ShapeDtypeStruct((_R, _C), jnp.float32),
        compiler_params=pltpu.CompilerParams(
            dimension_semantics=("arbitrary",)),
    )(idx)
    return out


# speedup vs baseline: 1.0898x; 1.0898x over previous
import jax
import jax.numpy as jnp
from jax.experimental import pallas as pl
from jax.experimental.pallas import tpu as pltpu

_R, _C = 128, 100000
_BC = 8192
_NB = pl.cdiv(_C, _BC)

_G_CACHE = None


def _gumbel_const():
    global _G_CACHE
    if _G_CACHE is None:
        _G_CACHE = jax.block_until_ready(
            jax.random.gumbel(jax.random.key(42), (_R, _C), jnp.float32))
    return _G_CACHE


def _argmax_body(x_ref, g_ref, idx_ref, m_ref):
    j = pl.program_id(0)

    @pl.when(j == 0)
    def _():
        m_ref[:] = jnp.full((_R, 1), -jnp.inf, jnp.float32)
        idx_ref[:] = jnp.zeros((_R, 1), jnp.int32)

    x = x_ref[:] + g_ref[:]
    cols = jax.lax.broadcasted_iota(jnp.int32, (_R, _BC), 1) + j * _BC
    x = jnp.where(cols < _C, x, -jnp.inf)
    bm = jnp.max(x, axis=1, keepdims=True)
    ba = jnp.min(jnp.where(x == bm, cols, _C), axis=1, keepdims=True)
    better = bm > m_ref[:]
    idx_ref[:] = jnp.where(better, ba, idx_ref[:]).astype(jnp.int32)
    m_ref[:] = jnp.where(better, bm, m_ref[:])


def _onehot_body(idx_ref, o_ref):
    j = pl.program_id(0)
    cols = jax.lax.broadcasted_iota(jnp.int32, (_R, _BC), 1) + j * _BC
    o_ref[:] = (cols == idx_ref[:]).astype(jnp.float32)


def kernel(logits):
    g = _gumbel_const()
    idx = pl.pallas_call(
        _argmax_body,
        grid=(_NB,),
        in_specs=[pl.BlockSpec((_R, _BC), lambda j: (0, j)),
                  pl.BlockSpec((_R, _BC), lambda j: (0, j))],
        out_specs=pl.BlockSpec((_R, 1), lambda j: (0, 0)),
        out_shape=jax.ShapeDtypeStruct((_R, 1), jnp.int32),
        scratch_shapes=[pltpu.VMEM((_R, 1), jnp.float32)],
    )(logits, g)
    out = pl.pallas_call(
        _onehot_body,
        grid=(_NB,),
        in_specs=[pl.BlockSpec((_R, 1), lambda j: (0, 0))],
        out_specs=pl.BlockSpec((_R, _BC), lambda j: (0, j)),
        out_shape=jax.ShapeDtypeStruct((_R, _C), jnp.float32),
        compiler_params=pltpu.CompilerParams(
            dimension_semantics=("arbitrary",)),
    )(idx)
    return out


# speedup vs baseline: 1.0905x; 1.0006x over previous
import numpy as np
import jax
import jax.numpy as jnp
from jax import lax
from jax.experimental import pallas as pl
from jax.experimental.pallas import tpu as pltpu

_R, _C = 128, 100000
_BC = 2048
_NB = pl.cdiv(_C, _BC)

_KS0 = 0
_KS1 = 42
_KS2 = _KS0 ^ _KS1 ^ 0x1BD11BDA
_ROT_A = (13, 15, 26, 6)
_ROT_B = (17, 29, 16, 24)
_TINY = np.float32(np.finfo(np.float32).tiny)


def _i32(v):
    return jnp.int32(np.uint32(v).view(np.int32))


def _rotl(x, d):
    return lax.shift_left(x, jnp.int32(d)) | lax.shift_right_logical(
        x, jnp.int32(32 - d))


def _threefry_rounds(x0, x1, rots):
    for r in rots:
        x0 = x0 + x1
        x1 = _rotl(x1, r)
        x1 = x0 ^ x1
    return x0, x1


def _gumbel_of_flat(f):
    x0 = jnp.zeros_like(f) + _i32(_KS0)
    x1 = f + _i32(_KS1)
    x0, x1 = _threefry_rounds(x0, x1, _ROT_A)
    x0, x1 = x0 + _i32(_KS1), x1 + _i32(_KS2 + 1)
    x0, x1 = _threefry_rounds(x0, x1, _ROT_B)
    x0, x1 = x0 + _i32(_KS2), x1 + _i32(_KS0 + 2)
    x0, x1 = _threefry_rounds(x0, x1, _ROT_A)
    x0, x1 = x0 + _i32(_KS0), x1 + _i32(_KS1 + 3)
    x0, x1 = _threefry_rounds(x0, x1, _ROT_B)
    x0, x1 = x0 + _i32(_KS1), x1 + _i32(_KS2 + 4)
    x0, x1 = _threefry_rounds(x0, x1, _ROT_A)
    x0, x1 = x0 + _i32(_KS2), x1 + _i32(_KS0 + 5)
    bits = x0 ^ x1
    fb = lax.shift_right_logical(bits, jnp.int32(9)) | _i32(0x3F800000)
    floats = lax.bitcast_convert_type(fb, jnp.float32) - jnp.float32(1.0)
    u = jnp.maximum(jnp.float32(_TINY),
                    floats * jnp.float32(1.0 - _TINY) + jnp.float32(_TINY))
    return -jnp.log(-jnp.log(u))


def _argmax_body(x_ref, idx_ref, m_ref):
    j = pl.program_id(0)

    @pl.when(j == 0)
    def _():
        m_ref[:] = jnp.full((_R, 1), -jnp.inf, jnp.float32)
        idx_ref[:] = jnp.zeros((_R, 1), jnp.int32)

    cols = jax.lax.broadcasted_iota(jnp.int32, (_R, _BC), 1) + j * _BC
    rows = jax.lax.broadcasted_iota(jnp.int32, (_R, _BC), 0)
    flat = rows * _C + cols
    g = _gumbel_of_flat(flat)
    x = x_ref[:] + g
    x = jnp.where(cols < _C, x, -jnp.inf)
    bm = jnp.max(x, axis=1, keepdims=True)
    ba = jnp.min(jnp.where(x == bm, cols, _C), axis=1, keepdims=True)
    better = bm > m_ref[:]
    idx_ref[:] = jnp.where(better, ba, idx_ref[:]).astype(jnp.int32)
    m_ref[:] = jnp.where(better, bm, m_ref[:])


def _onehot_body(idx_ref, o_ref):
    j = pl.program_id(0)
    cols = jax.lax.broadcasted_iota(jnp.int32, (_R, _BC), 1) + j * _BC
    o_ref[:] = (cols == idx_ref[:]).astype(jnp.float32)


def _threefry_argmax(logits):
    return pl.pallas_call(
        _argmax_body,
        grid=(_NB,),
        in_specs=[pl.BlockSpec((_R, _BC), lambda j: (0, j))],
        out_specs=pl.BlockSpec((_R, 1), lambda j: (0, 0)),
        out_shape=jax.ShapeDtypeStruct((_R, 1), jnp.int32),
        scratch_shapes=[pltpu.VMEM((_R, 1), jnp.float32)],
    )(logits)


def _onehot(idx):
    return pl.pallas_call(
        _onehot_body,
        grid=(_NB,),
        in_specs=[pl.BlockSpec((_R, 1), lambda j: (0, 0))],
        out_specs=pl.BlockSpec((_R, _BC), lambda j: (0, j)),
        out_shape=jax.ShapeDtypeStruct((_R, _C), jnp.float32),
    )(idx)


def kernel(logits):
    idx = _threefry_argmax(logits)
    return _onehot(idx)
